# baseline (speedup 1.0000x reference)
import functools

import jax
import jax.numpy as jnp
from jax.experimental import pallas as pl
from jax.experimental.pallas import tpu as pltpu

_N_TILE = 1024


def _chamfer_kernel(a_ref, bt_ref, fwd_ref, bwd_ref):
    i = pl.program_id(1)
    a = a_ref[0]
    bt = bt_ref[0]
    acc = None
    for k in range(3):
        diff = a[:, k : k + 1] - bt[k : k + 1, :]
        sq = diff * diff
        acc = sq if acc is None else acc + sq
    fwd_ref[0, 0, :] = jnp.min(acc, axis=1)
    bwd_tile = jnp.min(acc, axis=0)

    @pl.when(i == 0)
    def _():
        bwd_ref[0, 0, :] = bwd_tile

    @pl.when(i != 0)
    def _():
        bwd_ref[0, 0, :] = jnp.minimum(bwd_ref[0, 0, :], bwd_tile)


@jax.jit
def kernel(yhat, y):
    B, N, D = yhat.shape
    M = y.shape[1]
    y_t = jnp.transpose(y, (0, 2, 1))
    n_tiles = N // _N_TILE

    fwd, bwd = pl.pallas_call(
        _chamfer_kernel,
        grid=(B, n_tiles),
        in_specs=[
            pl.BlockSpec((1, _N_TILE, D), lambda b, i: (b, i, 0)),
            pl.BlockSpec((1, D, M), lambda b, i: (b, 0, 0)),
        ],
        out_specs=[
            pl.BlockSpec((1, 1, _N_TILE), lambda b, i: (b * n_tiles + i, 0, 0)),
            pl.BlockSpec((1, 1, M), lambda b, i: (b, 0, 0)),
        ],
        out_shape=[
            jax.ShapeDtypeStruct((B * n_tiles, 1, _N_TILE), jnp.float32),
            jax.ShapeDtypeStruct((B, 1, M), jnp.float32),
        ],
        compiler_params=pltpu.CompilerParams(
            dimension_semantics=("parallel", "arbitrary"),
        ),
    )(yhat, y_t)

    loss = jnp.mean(
        jnp.mean(fwd.reshape(B, N), axis=1) + jnp.mean(bwd.reshape(B, M), axis=1)
    )
    return jnp.sqrt(0.5 * loss)


# speedup vs baseline: 1.0027x; 1.0027x over previous
import jax
import jax.numpy as jnp
from jax.experimental import pallas as pl
from jax.experimental.pallas import tpu as pltpu

_N_TILE = 1024


def _chamfer_kernel(a_ref, bt_ref, fwd_ref, bwd_ref):
    i = pl.program_id(1)
    a = a_ref[0]
    bt = bt_ref[0]

    d = None
    for k in range(3):
        diff = a[:, k : k + 1] - bt[k : k + 1, :]
        sq = diff * diff
        d = sq if d is None else d + sq

    fwd_ref[0, 0, 0, :] = jnp.min(d, axis=1)
    bwd_tile = jnp.min(d, axis=0)

    @pl.when(i == 0)
    def _():
        bwd_ref[0, 0, :] = bwd_tile

    @pl.when(i != 0)
    def _():
        bwd_ref[0, 0, :] = jnp.minimum(bwd_ref[0, 0, :], bwd_tile)


@jax.jit
def kernel(yhat, y):
    B, N, D = yhat.shape
    M = y.shape[1]
    y_t = jnp.transpose(y, (0, 2, 1))
    n_tiles = N // _N_TILE

    fwd, bwd = pl.pallas_call(
        _chamfer_kernel,
        grid=(B, n_tiles),
        in_specs=[
            pl.BlockSpec((1, _N_TILE, D), lambda b, i: (b, i, 0)),
            pl.BlockSpec((1, D, M), lambda b, i: (b, 0, 0)),
        ],
        out_specs=[
            pl.BlockSpec((1, 1, 1, _N_TILE), lambda b, i: (b, i, 0, 0)),
            pl.BlockSpec((1, 1, M), lambda b, i: (b, 0, 0)),
        ],
        out_shape=[
            jax.ShapeDtypeStruct((B, n_tiles, 1, _N_TILE), jnp.float32),
            jax.ShapeDtypeStruct((B, 1, M), jnp.float32),
        ],
        compiler_params=pltpu.CompilerParams(
            dimension_semantics=("parallel", "arbitrary"),
        ),
    )(yhat, y_t)

    loss = jnp.mean(
        jnp.mean(fwd.reshape(B, N), axis=1) + jnp.mean(bwd.reshape(B, M), axis=1)
    )
    return jnp.sqrt(0.5 * loss)


# speedup vs baseline: 1.5278x; 1.5236x over previous
import jax
import jax.numpy as jnp
from jax.experimental import pallas as pl
from jax.experimental.pallas import tpu as pltpu

_N_TILE = 1024


def _chamfer_kernel(a_ref, bt_ref, fwd_ref, bwd_ref):
    i = pl.program_id(1)
    a = a_ref[0]
    bt = bt_ref[0]

    ab = jax.lax.dot_general(
        a, bt, (((1,), (0,)), ((), ())), preferred_element_type=jnp.float32
    )
    a2 = jnp.sum(a * a, axis=1, keepdims=True)
    b2 = jnp.sum(bt * bt, axis=0, keepdims=True)
    d = jnp.maximum(a2 + b2 - 2.0 * ab, 0.0)

    fwd_ref[0, 0, 0, :] = jnp.min(d, axis=1)
    bwd_tile = jnp.min(d, axis=0)

    @pl.when(i == 0)
    def _():
        bwd_ref[0, 0, :] = bwd_tile

    @pl.when(i != 0)
    def _():
        bwd_ref[0, 0, :] = jnp.minimum(bwd_ref[0, 0, :], bwd_tile)


@jax.jit
def kernel(yhat, y):
    B, N, D = yhat.shape
    M = y.shape[1]
    y_t = jnp.transpose(y, (0, 2, 1))
    n_tiles = N // _N_TILE

    fwd, bwd = pl.pallas_call(
        _chamfer_kernel,
        grid=(B, n_tiles),
        in_specs=[
            pl.BlockSpec((1, _N_TILE, D), lambda b, i: (b, i, 0)),
            pl.BlockSpec((1, D, M), lambda b, i: (b, 0, 0)),
        ],
        out_specs=[
            pl.BlockSpec((1, 1, 1, _N_TILE), lambda b, i: (b, i, 0, 0)),
            pl.BlockSpec((1, 1, M), lambda b, i: (b, 0, 0)),
        ],
        out_shape=[
            jax.ShapeDtypeStruct((B, n_tiles, 1, _N_TILE), jnp.float32),
            jax.ShapeDtypeStruct((B, 1, M), jnp.float32),
        ],
        compiler_params=pltpu.CompilerParams(
            dimension_semantics=("parallel", "arbitrary"),
        ),
    )(yhat, y_t)

    loss = jnp.mean(
        jnp.mean(fwd.reshape(B, N), axis=1) + jnp.mean(bwd.reshape(B, M), axis=1)
    )
    return jnp.sqrt(0.5 * loss)


# speedup vs baseline: 1.6638x; 1.0890x over previous
import jax
import jax.numpy as jnp
from jax.experimental import pallas as pl
from jax.experimental.pallas import tpu as pltpu

_N_TILE = 1024


def _chamfer_kernel(a_ref, bt_ref, fwd_ref, bwd_ref):
    i = pl.program_id(1)
    n_tiles = pl.num_programs(1)
    a = a_ref[0]
    bt = bt_ref[0]

    ab = jax.lax.dot_general(
        a, bt, (((1,), (0,)), ((), ())), preferred_element_type=jnp.float32
    )
    a2 = jnp.sum(a * a, axis=1, keepdims=True)
    b2 = jnp.sum(bt * bt, axis=0, keepdims=True)

    s = -2.0 * ab
    e = s + b2
    f = s + a2

    fwd_ref[0, 0, 0, :] = jnp.maximum(jnp.min(e, axis=1) + a2[:, 0], 0.0)
    bwd_tile = jnp.min(f, axis=0)

    @pl.when(i == 0)
    def _():
        bwd_ref[0, 0, :] = bwd_tile

    @pl.when(i != 0)
    def _():
        bwd_ref[0, 0, :] = jnp.minimum(bwd_ref[0, 0, :], bwd_tile)

    @pl.when(i == n_tiles - 1)
    def _():
        bwd_ref[0, 0, :] = jnp.maximum(bwd_ref[0, 0, :] + b2[0, :], 0.0)


@jax.jit
def kernel(yhat, y):
    B, N, D = yhat.shape
    M = y.shape[1]
    y_t = jnp.transpose(y, (0, 2, 1))
    n_tiles = N // _N_TILE

    fwd, bwd = pl.pallas_call(
        _chamfer_kernel,
        grid=(B, n_tiles),
        in_specs=[
            pl.BlockSpec((1, _N_TILE, D), lambda b, i: (b, i, 0)),
            pl.BlockSpec((1, D, M), lambda b, i: (b, 0, 0)),
        ],
        out_specs=[
            pl.BlockSpec((1, 1, 1, _N_TILE), lambda b, i: (b, i, 0, 0)),
            pl.BlockSpec((1, 1, M), lambda b, i: (b, 0, 0)),
        ],
        out_shape=[
            jax.ShapeDtypeStruct((B, n_tiles, 1, _N_TILE), jnp.float32),
            jax.ShapeDtypeStruct((B, 1, M), jnp.float32),
        ],
        compiler_params=pltpu.CompilerParams(
            dimension_semantics=("parallel", "arbitrary"),
        ),
    )(yhat, y_t)

    loss = jnp.mean(
        jnp.mean(fwd.reshape(B, N), axis=1) + jnp.mean(bwd.reshape(B, M), axis=1)
    )
    return jnp.sqrt(0.5 * loss)


# speedup vs baseline: 1.7814x; 1.0707x over previous
import jax
import jax.numpy as jnp
from jax.experimental import pallas as pl
from jax.experimental.pallas import tpu as pltpu

_N_TILE = 1024


def _chamfer_kernel(a_ref, bt_ref, fwd_ref, bwd_ref):
    i = pl.program_id(1)
    n_tiles = pl.num_programs(1)
    a = a_ref[0]
    bt = bt_ref[0]

    a2 = jnp.sum(a * a, axis=1, keepdims=True)
    b2 = jnp.sum(bt * bt, axis=0, keepdims=True)

    s = jax.lax.dot_general(
        a, -2.0 * bt, (((1,), (0,)), ((), ())), preferred_element_type=jnp.float32
    )
    e = s + b2
    f = s + a2

    fwd_ref[0, 0, 0, :] = jnp.maximum(jnp.min(e, axis=1) + a2[:, 0], 0.0)
    bwd_tile = jnp.min(f, axis=0)

    @pl.when(i == 0)
    def _():
        bwd_ref[0, 0, :] = bwd_tile

    @pl.when(i != 0)
    def _():
        bwd_ref[0, 0, :] = jnp.minimum(bwd_ref[0, 0, :], bwd_tile)

    @pl.when(i == n_tiles - 1)
    def _():
        bwd_ref[0, 0, :] = jnp.maximum(bwd_ref[0, 0, :] + b2[0, :], 0.0)


@jax.jit
def kernel(yhat, y):
    B, N, D = yhat.shape
    M = y.shape[1]
    y_t = jnp.transpose(y, (0, 2, 1))
    n_tiles = N // _N_TILE

    fwd, bwd = pl.pallas_call(
        _chamfer_kernel,
        grid=(B, n_tiles),
        in_specs=[
            pl.BlockSpec((1, _N_TILE, D), lambda b, i: (b, i, 0)),
            pl.BlockSpec((1, D, M), lambda b, i: (b, 0, 0)),
        ],
        out_specs=[
            pl.BlockSpec((1, 1, 1, _N_TILE), lambda b, i: (b, i, 0, 0)),
            pl.BlockSpec((1, 1, M), lambda b, i: (b, 0, 0)),
        ],
        out_shape=[
            jax.ShapeDtypeStruct((B, n_tiles, 1, _N_TILE), jnp.float32),
            jax.ShapeDtypeStruct((B, 1, M), jnp.float32),
        ],
        compiler_params=pltpu.CompilerParams(
            dimension_semantics=("parallel", "arbitrary"),
        ),
    )(yhat, y_t)

    loss = jnp.mean(
        jnp.mean(fwd.reshape(B, N), axis=1) + jnp.mean(bwd.reshape(B, M), axis=1)
    )
    return jnp.sqrt(0.5 * loss)


# speedup vs baseline: 2.2177x; 1.2449x over previous
import jax
import jax.numpy as jnp
from jax.experimental import pallas as pl
from jax.experimental.pallas import tpu as pltpu

_N_TILE = 1024


def _chamfer_kernel(a_ref, bt_ref, fwd_ref, bwd_ref):
    i = pl.program_id(1)
    n_tiles = pl.num_programs(1)
    a = a_ref[0]
    bt = bt_ref[0]

    a2 = jnp.sum(a * a, axis=1, keepdims=True)
    b2 = jnp.sum(bt * bt, axis=0, keepdims=True)

    s = jax.lax.dot_general(
        a, -2.0 * bt, (((1,), (0,)), ((), ())), preferred_element_type=jnp.float32
    )
    e = s + b2
    f = s + a2

    fwd_ref[0, :, :] = jnp.maximum(jnp.min(e, axis=1, keepdims=True) + a2, 0.0)
    bwd_tile = jnp.min(f, axis=0)

    @pl.when(i == 0)
    def _():
        bwd_ref[0, 0, :] = bwd_tile

    @pl.when(i != 0)
    def _():
        bwd_ref[0, 0, :] = jnp.minimum(bwd_ref[0, 0, :], bwd_tile)

    @pl.when(i == n_tiles - 1)
    def _():
        bwd_ref[0, 0, :] = jnp.maximum(bwd_ref[0, 0, :] + b2[0, :], 0.0)


@jax.jit
def kernel(yhat, y):
    B, N, D = yhat.shape
    M = y.shape[1]
    y_t = jnp.transpose(y, (0, 2, 1))
    n_tiles = N // _N_TILE

    fwd, bwd = pl.pallas_call(
        _chamfer_kernel,
        grid=(B, n_tiles),
        in_specs=[
            pl.BlockSpec((1, _N_TILE, D), lambda b, i: (b, i, 0)),
            pl.BlockSpec((1, D, M), lambda b, i: (b, 0, 0)),
        ],
        out_specs=[
            pl.BlockSpec((1, _N_TILE, 1), lambda b, i: (b * n_tiles + i, 0, 0)),
            pl.BlockSpec((1, 1, M), lambda b, i: (b, 0, 0)),
        ],
        out_shape=[
            jax.ShapeDtypeStruct((B * n_tiles, _N_TILE, 1), jnp.float32),
            jax.ShapeDtypeStruct((B, 1, M), jnp.float32),
        ],
        compiler_params=pltpu.CompilerParams(
            dimension_semantics=("parallel", "arbitrary"),
        ),
    )(yhat, y_t)

    loss = jnp.mean(
        jnp.mean(fwd.reshape(B, N), axis=1) + jnp.mean(bwd.reshape(B, M), axis=1)
    )
    return jnp.sqrt(0.5 * loss)
